# baseline (speedup 1.0000x reference)
import functools

import jax
import jax.numpy as jnp
from jax import lax
from jax.experimental import pallas as pl
from jax.experimental.pallas import tpu as pltpu
from jax.experimental.pallas import tpu_sc as plsc

NUM_TOP = 300
B, N, C = 16, 20000, 80
NBLK = 4000
KPAD = 304
NGAT = 384


def _reduce_body(x_ref, s_ref, l_ref):
    x = x_ref[0]
    iota = lax.broadcasted_iota(jnp.int32, (NBLK, C), 1)
    big = jnp.int32(C)
    m1 = jnp.max(x, axis=1)
    a1 = jnp.min(jnp.where(x == m1[:, None], iota, big), axis=1)
    masked = jnp.where(iota == a1[:, None], -jnp.inf, x)
    m2 = jnp.max(masked, axis=1)
    a2 = jnp.min(jnp.where((x == m2[:, None]) & (iota != a1[:, None]), iota, big),
                 axis=1)
    s1 = jax.nn.sigmoid(m1)
    s2 = jax.nn.sigmoid(m2)
    label = jnp.where((s2 == s1) & (a2 < a1), a2, a1)
    s_ref[0, 0] = s1
    l_ref[0, 0] = label


def _scores_labels(pred_logits):
    nblocks = N // NBLK
    grid = (B, nblocks)
    s, l = pl.pallas_call(
        _reduce_body,
        grid=grid,
        in_specs=[pl.BlockSpec((1, NBLK, C), lambda b, i: (b, i, 0))],
        out_specs=[
            pl.BlockSpec((1, 1, NBLK), lambda b, i: (b * nblocks + i, 0, 0)),
            pl.BlockSpec((1, 1, NBLK), lambda b, i: (b * nblocks + i, 0, 0)),
        ],
        out_shape=[
            jax.ShapeDtypeStruct((B * nblocks, 1, NBLK), jnp.float32),
            jax.ShapeDtypeStruct((B * nblocks, 1, NBLK), jnp.int32),
        ],
    )(pred_logits)
    return s.reshape(B, N), l.reshape(B, N)


def _topk_body(scores_hbm, labels_hbm, boxes_hbm, sizes_hbm,
               boxes_out, scores_out, labels_out,
               srow, lrow, hist, totals, gt_idx, eq_idx, gkeys, gidxs,
               sorted_idx, out_s, out_l, brows, outb, sz):
    iota = lax.iota(jnp.int32, 16)
    wid = lax.axis_index("s") * 2 + lax.axis_index("c")
    b = wid
    nvec = N // 16

    @pl.when(wid < B)
    def _():
        pltpu.sync_copy(scores_hbm.at[b], srow)
        pltpu.sync_copy(labels_hbm.at[b], lrow)
        pltpu.sync_copy(boxes_hbm.at[b], brows)
        pltpu.sync_copy(sizes_hbm, sz)

        ones = jnp.ones((16,), jnp.int32)

        def select_pass(p, prefix, k_rem):
            sh = 24 - 8 * p

            def zh(i, _):
                hist[pl.ds(i * 16, 16)] = jnp.zeros((16,), jnp.int32)
                return 0
            lax.fori_loop(0, 256, zh, 0)

            def hb(i, _):
                kv = lax.bitcast_convert_type(srow[pl.ds(i * 16, 16)], jnp.int32)
                d = ((kv >> sh) & 0xFF).astype(jnp.int32)
                if p == 0:
                    match = iota >= 0
                else:
                    match = (kv >> (sh + 8)) == prefix
                hidx = d * 16 + iota
                cur = plsc.load_gather(hist, [hidx])
                plsc.store_scatter(hist, [hidx],
                                   cur + jnp.where(match, 1, 0))
                return 0
            lax.fori_loop(0, nvec, hb, 0)

            def tb(g, _):
                acc = jnp.zeros((16,), jnp.int32)
                for j in range(16):
                    acc = acc + plsc.load_gather(hist, [(g * 16 + iota) * 16 + j])
                totals[pl.ds(g * 16, 16)] = acc
                return 0
            lax.fori_loop(0, 16, tb, 0)

            def sb(gi, carry):
                cum, cnt = carry
                g = 15 - gi
                tg = totals[pl.ds(g * 16, 16)]
                sfx = lax.rev(plsc.cumsum(lax.rev(tg, (0,))), (0,)) + cum
                cnt = cnt + plsc.all_reduce_population_count(sfx >= k_rem)[0]
                cum = cum + jnp.sum(tg)
                return cum, cnt
            _, pop = lax.fori_loop(0, 16, sb, (jnp.int32(0), jnp.int32(0)))
            dstar = pop - 1

            def s2b(g, acc):
                tg = totals[pl.ds(g * 16, 16)]
                return acc + jnp.sum(jnp.where((g * 16 + iota) >= dstar, tg, 0))
            s_dstar = lax.fori_loop(0, 16, s2b, jnp.int32(0))
            t_dstar = totals[pl.ds(dstar, 16)][0]
            k_rem = k_rem - (s_dstar - t_dstar)
            prefix = (prefix << 8) | dstar
            return prefix, k_rem

        prefix = jnp.int32(0)
        k_rem = jnp.int32(NUM_TOP)
        for p in range(4):
            prefix, k_rem = select_pass(p, prefix, k_rem)

        def cb(i, carry):
            ogt, oeq = carry
            kv = lax.bitcast_convert_type(srow[pl.ds(i * 16, 16)], jnp.int32)
            idxv = i * 16 + iota
            mgt = kv > prefix
            meq = kv == prefix
            plsc.store_compressed(gt_idx.at[pl.ds(ogt, 16)], idxv, mask=mgt)
            plsc.store_compressed(eq_idx.at[pl.ds(jnp.minimum(oeq, KPAD), 16)],
                                  idxv, mask=meq)
            ogt = ogt + plsc.all_reduce_population_count(mgt)[0]
            oeq = oeq + plsc.all_reduce_population_count(meq)[0]
            return ogt, oeq
        n_gt, _ = lax.fori_loop(0, nvec, cb, (jnp.int32(0), jnp.int32(0)))

        def lb(v, _):
            p16 = v * 16 + iota
            valid = p16 < n_gt
            gidx = jnp.where(valid, gt_idx[pl.ds(v * 16, 16)], 0)
            keys = plsc.load_gather(srow, [gidx])
            gkeys[pl.ds(v * 16, 16)] = jnp.where(valid, keys, jnp.float32(-1.0))
            gidxs[pl.ds(v * 16, 16)] = jnp.where(valid, gidx, N + p16)
            return 0
        lax.fori_loop(0, KPAD // 16, lb, 0)

        def rt(t, _):
            kt = gkeys[pl.ds(t * 16, 16)]
            it = gidxs[pl.ds(t * 16, 16)]

            def rs(sv, cnt):
                kvec = gkeys[pl.ds(sv * 16, 16)]
                ivec = gidxs[pl.ds(sv * 16, 16)]
                for l in range(16):
                    kj = kvec[l]
                    ij = ivec[l]
                    beat = (kj > kt) | ((kj == kt) & (ij < it))
                    cnt = cnt + jnp.where(beat, 1, 0)
                return cnt
            rank = lax.fori_loop(0, KPAD // 16, rs,
                                 jnp.zeros((16,), jnp.int32))
            plsc.store_scatter(sorted_idx, [rank], it)
            return 0
        lax.fori_loop(0, KPAD // 16, rt, 0)

        def eb(v, _):
            p16 = v * 16 + iota
            cur = sorted_idx[pl.ds(v * 16, 16)]
            ev = plsc.load_gather(eq_idx, [jnp.clip(p16 - n_gt, 0, KPAD - 1)])
            sorted_idx[pl.ds(v * 16, 16)] = jnp.where(p16 >= n_gt, ev, cur)
            return 0
        lax.fori_loop(0, KPAD // 16, eb, 0)

        def ob(v, _):
            sic = jnp.clip(sorted_idx[pl.ds(v * 16, 16)], 0, N - 1)
            out_s[pl.ds(v * 16, 16)] = plsc.load_gather(srow, [sic])
            out_l[pl.ds(v * 16, 16)] = plsc.load_gather(lrow, [sic])
            return 0
        lax.fori_loop(0, KPAD // 16, ob, 0)

        szlo = sz[pl.ds(0, 16)]
        szhi = sz[pl.ds(16, 16)]
        szsel = jnp.where(b < 8, szlo, szhi)
        lane = jnp.broadcast_to((2 * b) % 16, (16,))
        hf = szsel.at[lane].get(mode="promise_in_bounds")[0]
        wf = szsel.at[lane + 1].get(mode="promise_in_bounds")[0]
        half = jnp.float32(0.5)

        def bb(v, _):
            p16 = v * 16 + iota
            kq = p16 >> 2
            comp = p16 & 3
            base = comp & 1
            qv = jnp.clip(plsc.load_gather(sorted_idx, [kq]), 0, N - 1)
            ctr = plsc.load_gather(brows, [qv * 4 + base])
            ext = plsc.load_gather(brows, [qv * 4 + base + 2])
            sgn = jnp.where(comp >= 2, half, -half)
            scl = jnp.where(base == 1, hf, wf)
            outb[pl.ds(v * 16, 16)] = (ctr + sgn * ext) * scl
            return 0
        lax.fori_loop(0, KPAD * 4 // 16, bb, 0)

        pltpu.sync_copy(outb, boxes_out.at[b])
        pltpu.sync_copy(out_s, scores_out.at[b])
        pltpu.sync_copy(out_l, labels_out.at[b])


def _topk_sc(scores, labels, boxes_flat, sizes):
    mesh = plsc.VectorSubcoreMesh(core_axis_name="c", subcore_axis_name="s")
    f = pl.kernel(
        _topk_body,
        out_type=[
            jax.ShapeDtypeStruct((B, KPAD * 4), jnp.float32),
            jax.ShapeDtypeStruct((B, KPAD), jnp.float32),
            jax.ShapeDtypeStruct((B, KPAD), jnp.int32),
        ],
        mesh=mesh,
        compiler_params=pltpu.CompilerParams(needs_layout_passes=False),
        scratch_types=[
            pltpu.VMEM((N,), jnp.float32),
            pltpu.VMEM((N,), jnp.int32),
            pltpu.VMEM((4096,), jnp.int32),
            pltpu.VMEM((272,), jnp.int32),
            pltpu.VMEM((320,), jnp.int32),
            pltpu.VMEM((KPAD + 32,), jnp.int32),
            pltpu.VMEM((KPAD,), jnp.float32),
            pltpu.VMEM((KPAD,), jnp.int32),
            pltpu.VMEM((KPAD,), jnp.int32),
            pltpu.VMEM((KPAD,), jnp.float32),
            pltpu.VMEM((KPAD,), jnp.int32),
            pltpu.VMEM((N * 4,), jnp.float32),
            pltpu.VMEM((KPAD * 4,), jnp.float32),
            pltpu.VMEM((2 * B,), jnp.float32),
        ],
    )
    return f(scores, labels, boxes_flat, sizes)


def kernel(pred_logits, pred_boxes, orig_target_sizes):
    scores, labels = _scores_labels(pred_logits)
    sizes_f = orig_target_sizes.astype(jnp.float32).reshape(2 * B)
    bflat, s, l = _topk_sc(scores, labels, pred_boxes.reshape(B, N * 4),
                           sizes_f)
    boxes = bflat.reshape(B, KPAD, 4)[:, :NUM_TOP]
    return boxes, s[:, :NUM_TOP], l[:, :NUM_TOP]


# speedup vs baseline: 1.3855x; 1.3855x over previous
import functools

import jax
import jax.numpy as jnp
from jax import lax
from jax.experimental import pallas as pl
from jax.experimental.pallas import tpu as pltpu
from jax.experimental.pallas import tpu_sc as plsc

NUM_TOP = 300
B, N, C = 16, 20000, 80
NBLK = 4000
KPAD = 304
NGAT = 384


def _reduce_body(x_ref, s_ref, l_ref):
    x = x_ref[0]
    s = jax.nn.sigmoid(x)
    s_ref[0, 0] = jnp.max(s, axis=1)
    l_ref[0, 0] = jnp.argmax(s, axis=1).astype(jnp.int32)


def _scores_labels(pred_logits):
    nblocks = N // NBLK
    grid = (B, nblocks)
    s, l = pl.pallas_call(
        _reduce_body,
        grid=grid,
        in_specs=[pl.BlockSpec((1, NBLK, C), lambda b, i: (b, i, 0))],
        out_specs=[
            pl.BlockSpec((1, 1, NBLK), lambda b, i: (b * nblocks + i, 0, 0)),
            pl.BlockSpec((1, 1, NBLK), lambda b, i: (b * nblocks + i, 0, 0)),
        ],
        out_shape=[
            jax.ShapeDtypeStruct((B * nblocks, 1, NBLK), jnp.float32),
            jax.ShapeDtypeStruct((B * nblocks, 1, NBLK), jnp.int32),
        ],
    )(pred_logits)
    return s.reshape(B, N), l.reshape(B, N)


def _topk_body(scores_hbm, labels_hbm, boxes_hbm, sizes_hbm,
               boxes_out, scores_out, labels_out,
               srow, lrow, hist, totals, gt_idx, eq_idx, gkeys, gidxs,
               sorted_idx, out_s, out_l, brows, outb, sz):
    iota = lax.iota(jnp.int32, 16)
    wid = lax.axis_index("s") * 2 + lax.axis_index("c")
    b = wid
    nvec = N // 16

    @pl.when(wid < B)
    def _():
        pltpu.sync_copy(scores_hbm.at[b], srow)
        pltpu.sync_copy(labels_hbm.at[b], lrow)
        pltpu.sync_copy(boxes_hbm.at[b], brows)
        pltpu.sync_copy(sizes_hbm, sz)

        ones = jnp.ones((16,), jnp.int32)

        def select_pass(p, prefix, k_rem):
            sh = 24 - 8 * p

            def zh(i, _):
                hist[pl.ds(i * 16, 16)] = jnp.zeros((16,), jnp.int32)
                return 0
            lax.fori_loop(0, 256, zh, 0)

            def hb(i, _):
                kv = lax.bitcast_convert_type(srow[pl.ds(i * 16, 16)], jnp.int32)
                d = ((kv >> sh) & 0xFF).astype(jnp.int32)
                if p == 0:
                    match = iota >= 0
                else:
                    match = (kv >> (sh + 8)) == prefix
                hidx = d * 16 + iota
                cur = plsc.load_gather(hist, [hidx])
                plsc.store_scatter(hist, [hidx],
                                   cur + jnp.where(match, 1, 0))
                return 0
            lax.fori_loop(0, nvec, hb, 0)

            def tb(g, _):
                acc = jnp.zeros((16,), jnp.int32)
                for j in range(16):
                    acc = acc + plsc.load_gather(hist, [(g * 16 + iota) * 16 + j])
                totals[pl.ds(g * 16, 16)] = acc
                return 0
            lax.fori_loop(0, 16, tb, 0)

            def sb(gi, carry):
                cum, cnt = carry
                g = 15 - gi
                tg = totals[pl.ds(g * 16, 16)]
                sfx = lax.rev(plsc.cumsum(lax.rev(tg, (0,))), (0,)) + cum
                cnt = cnt + plsc.all_reduce_population_count(sfx >= k_rem)[0]
                cum = cum + jnp.sum(tg)
                return cum, cnt
            _, pop = lax.fori_loop(0, 16, sb, (jnp.int32(0), jnp.int32(0)))
            dstar = pop - 1

            def s2b(g, acc):
                tg = totals[pl.ds(g * 16, 16)]
                return acc + jnp.sum(jnp.where((g * 16 + iota) >= dstar, tg, 0))
            s_dstar = lax.fori_loop(0, 16, s2b, jnp.int32(0))
            t_dstar = totals[pl.ds(dstar, 16)][0]
            k_rem = k_rem - (s_dstar - t_dstar)
            prefix = (prefix << 8) | dstar
            return prefix, k_rem

        prefix = jnp.int32(0)
        k_rem = jnp.int32(NUM_TOP)
        for p in range(4):
            prefix, k_rem = select_pass(p, prefix, k_rem)

        def cb(i, carry):
            ogt, oeq = carry
            kv = lax.bitcast_convert_type(srow[pl.ds(i * 16, 16)], jnp.int32)
            idxv = i * 16 + iota
            mgt = kv > prefix
            meq = kv == prefix
            plsc.store_compressed(gt_idx.at[pl.ds(ogt, 16)], idxv, mask=mgt)
            plsc.store_compressed(eq_idx.at[pl.ds(jnp.minimum(oeq, KPAD), 16)],
                                  idxv, mask=meq)
            ogt = ogt + plsc.all_reduce_population_count(mgt)[0]
            oeq = oeq + plsc.all_reduce_population_count(meq)[0]
            return ogt, oeq
        n_gt, _ = lax.fori_loop(0, nvec, cb, (jnp.int32(0), jnp.int32(0)))

        def lb(v, _):
            p16 = v * 16 + iota
            valid = p16 < n_gt
            gidx = jnp.where(valid, gt_idx[pl.ds(v * 16, 16)], 0)
            keys = plsc.load_gather(srow, [gidx])
            gkeys[pl.ds(v * 16, 16)] = jnp.where(valid, keys, jnp.float32(-1.0))
            gidxs[pl.ds(v * 16, 16)] = jnp.where(valid, gidx, N + p16)
            return 0
        lax.fori_loop(0, KPAD // 16, lb, 0)

        def rt(t, _):
            kt = gkeys[pl.ds(t * 16, 16)]
            it = gidxs[pl.ds(t * 16, 16)]

            def rs(sv, cnt):
                kvec = gkeys[pl.ds(sv * 16, 16)]
                ivec = gidxs[pl.ds(sv * 16, 16)]
                for l in range(16):
                    kj = kvec[l]
                    ij = ivec[l]
                    beat = (kj > kt) | ((kj == kt) & (ij < it))
                    cnt = cnt + jnp.where(beat, 1, 0)
                return cnt
            rank = lax.fori_loop(0, KPAD // 16, rs,
                                 jnp.zeros((16,), jnp.int32))
            plsc.store_scatter(sorted_idx, [rank], it)
            return 0
        lax.fori_loop(0, KPAD // 16, rt, 0)

        def eb(v, _):
            p16 = v * 16 + iota
            cur = sorted_idx[pl.ds(v * 16, 16)]
            ev = plsc.load_gather(eq_idx, [jnp.clip(p16 - n_gt, 0, KPAD - 1)])
            sorted_idx[pl.ds(v * 16, 16)] = jnp.where(p16 >= n_gt, ev, cur)
            return 0
        lax.fori_loop(0, KPAD // 16, eb, 0)

        def ob(v, _):
            sic = jnp.clip(sorted_idx[pl.ds(v * 16, 16)], 0, N - 1)
            out_s[pl.ds(v * 16, 16)] = plsc.load_gather(srow, [sic])
            out_l[pl.ds(v * 16, 16)] = plsc.load_gather(lrow, [sic])
            return 0
        lax.fori_loop(0, KPAD // 16, ob, 0)

        szlo = sz[pl.ds(0, 16)]
        szhi = sz[pl.ds(16, 16)]
        szsel = jnp.where(b < 8, szlo, szhi)
        lane = jnp.broadcast_to((2 * b) % 16, (16,))
        hf = szsel.at[lane].get(mode="promise_in_bounds")[0]
        wf = szsel.at[lane + 1].get(mode="promise_in_bounds")[0]
        half = jnp.float32(0.5)

        def bb(v, _):
            p16 = v * 16 + iota
            kq = p16 >> 2
            comp = p16 & 3
            base = comp & 1
            qv = jnp.clip(plsc.load_gather(sorted_idx, [kq]), 0, N - 1)
            ctr = plsc.load_gather(brows, [qv * 4 + base])
            ext = plsc.load_gather(brows, [qv * 4 + base + 2])
            sgn = jnp.where(comp >= 2, half, -half)
            scl = jnp.where(base == 1, hf, wf)
            outb[pl.ds(v * 16, 16)] = (ctr + sgn * ext) * scl
            return 0
        lax.fori_loop(0, KPAD * 4 // 16, bb, 0)

        pltpu.sync_copy(outb, boxes_out.at[b])
        pltpu.sync_copy(out_s, scores_out.at[b])
        pltpu.sync_copy(out_l, labels_out.at[b])


def _topk_sc(scores, labels, boxes_flat, sizes):
    mesh = plsc.VectorSubcoreMesh(core_axis_name="c", subcore_axis_name="s")
    f = pl.kernel(
        _topk_body,
        out_type=[
            jax.ShapeDtypeStruct((B, KPAD * 4), jnp.float32),
            jax.ShapeDtypeStruct((B, KPAD), jnp.float32),
            jax.ShapeDtypeStruct((B, KPAD), jnp.int32),
        ],
        mesh=mesh,
        compiler_params=pltpu.CompilerParams(needs_layout_passes=False),
        scratch_types=[
            pltpu.VMEM((N,), jnp.float32),
            pltpu.VMEM((N,), jnp.int32),
            pltpu.VMEM((4096,), jnp.int32),
            pltpu.VMEM((272,), jnp.int32),
            pltpu.VMEM((320,), jnp.int32),
            pltpu.VMEM((KPAD + 32,), jnp.int32),
            pltpu.VMEM((KPAD,), jnp.float32),
            pltpu.VMEM((KPAD,), jnp.int32),
            pltpu.VMEM((KPAD,), jnp.int32),
            pltpu.VMEM((KPAD,), jnp.float32),
            pltpu.VMEM((KPAD,), jnp.int32),
            pltpu.VMEM((N * 4,), jnp.float32),
            pltpu.VMEM((KPAD * 4,), jnp.float32),
            pltpu.VMEM((2 * B,), jnp.float32),
        ],
    )
    return f(scores, labels, boxes_flat, sizes)


def kernel(pred_logits, pred_boxes, orig_target_sizes):
    scores, labels = _scores_labels(pred_logits)
    sizes_f = orig_target_sizes.astype(jnp.float32).reshape(2 * B)
    bflat, s, l = _topk_sc(scores, labels, pred_boxes.reshape(B, N * 4),
                           sizes_f)
    boxes = bflat.reshape(B, KPAD, 4)[:, :NUM_TOP]
    return boxes, s[:, :NUM_TOP], l[:, :NUM_TOP]


# speedup vs baseline: 2.0307x; 1.4657x over previous
import functools

import jax
import jax.numpy as jnp
from jax import lax
from jax.experimental import pallas as pl
from jax.experimental.pallas import tpu as pltpu
from jax.experimental.pallas import tpu_sc as plsc

NUM_TOP = 300
B, N, C = 16, 20000, 80
NBLK = 4000
KPAD = 304
NGAT = 384


def _reduce_body(x_ref, s_ref, l_ref):
    x = x_ref[0]
    s = jax.nn.sigmoid(x)
    st = s.T
    s_ref[0, 0] = jnp.max(st, axis=0)
    l_ref[0, 0] = jnp.argmax(st, axis=0).astype(jnp.int32)


def _scores_labels(pred_logits):
    nblocks = N // NBLK
    grid = (B, nblocks)
    s, l = pl.pallas_call(
        _reduce_body,
        grid=grid,
        in_specs=[pl.BlockSpec((1, NBLK, C), lambda b, i: (b, i, 0))],
        out_specs=[
            pl.BlockSpec((1, 1, NBLK), lambda b, i: (b * nblocks + i, 0, 0)),
            pl.BlockSpec((1, 1, NBLK), lambda b, i: (b * nblocks + i, 0, 0)),
        ],
        out_shape=[
            jax.ShapeDtypeStruct((B * nblocks, 1, NBLK), jnp.float32),
            jax.ShapeDtypeStruct((B * nblocks, 1, NBLK), jnp.int32),
        ],
    )(pred_logits)
    return s.reshape(B, N), l.reshape(B, N)


def _topk_body(scores_hbm, labels_hbm, boxes_hbm, sizes_hbm,
               boxes_out, scores_out, labels_out,
               srow, lrow, hist, totals, gt_idx, eq_idx, gkeys, gidxs,
               sorted_idx, out_s, out_l, brows, outb, sz):
    iota = lax.iota(jnp.int32, 16)
    wid = lax.axis_index("s") * 2 + lax.axis_index("c")
    b = wid
    nvec = N // 16

    @pl.when(wid < B)
    def _():
        pltpu.sync_copy(scores_hbm.at[b], srow)
        pltpu.sync_copy(labels_hbm.at[b], lrow)
        pltpu.sync_copy(boxes_hbm.at[b], brows)
        pltpu.sync_copy(sizes_hbm, sz)

        ones = jnp.ones((16,), jnp.int32)

        def select_pass(p, prefix, k_rem):
            sh = 24 - 8 * p

            def zh(i, _):
                hist[pl.ds(i * 16, 16)] = jnp.zeros((16,), jnp.int32)
                return 0
            lax.fori_loop(0, 256, zh, 0)

            def hb(i, _):
                kv = lax.bitcast_convert_type(srow[pl.ds(i * 16, 16)], jnp.int32)
                d = ((kv >> sh) & 0xFF).astype(jnp.int32)
                if p == 0:
                    match = iota >= 0
                else:
                    match = (kv >> (sh + 8)) == prefix
                hidx = d * 16 + iota
                cur = plsc.load_gather(hist, [hidx])
                plsc.store_scatter(hist, [hidx],
                                   cur + jnp.where(match, 1, 0))
                return 0
            lax.fori_loop(0, nvec, hb, 0)

            def tb(g, _):
                acc = jnp.zeros((16,), jnp.int32)
                for j in range(16):
                    acc = acc + plsc.load_gather(hist, [(g * 16 + iota) * 16 + j])
                totals[pl.ds(g * 16, 16)] = acc
                return 0
            lax.fori_loop(0, 16, tb, 0)

            def sb(gi, carry):
                cum, cnt = carry
                g = 15 - gi
                tg = totals[pl.ds(g * 16, 16)]
                sfx = lax.rev(plsc.cumsum(lax.rev(tg, (0,))), (0,)) + cum
                cnt = cnt + plsc.all_reduce_population_count(sfx >= k_rem)[0]
                cum = cum + jnp.sum(tg)
                return cum, cnt
            _, pop = lax.fori_loop(0, 16, sb, (jnp.int32(0), jnp.int32(0)))
            dstar = pop - 1

            def s2b(g, acc):
                tg = totals[pl.ds(g * 16, 16)]
                return acc + jnp.sum(jnp.where((g * 16 + iota) >= dstar, tg, 0))
            s_dstar = lax.fori_loop(0, 16, s2b, jnp.int32(0))
            t_dstar = totals[pl.ds(dstar, 16)][0]
            k_rem = k_rem - (s_dstar - t_dstar)
            prefix = (prefix << 8) | dstar
            return prefix, k_rem

        prefix = jnp.int32(0)
        k_rem = jnp.int32(NUM_TOP)
        for p in range(4):
            prefix, k_rem = select_pass(p, prefix, k_rem)

        def cb(i, carry):
            ogt, oeq = carry
            kv = lax.bitcast_convert_type(srow[pl.ds(i * 16, 16)], jnp.int32)
            idxv = i * 16 + iota
            mgt = kv > prefix
            meq = kv == prefix
            plsc.store_compressed(gt_idx.at[pl.ds(ogt, 16)], idxv, mask=mgt)
            plsc.store_compressed(eq_idx.at[pl.ds(jnp.minimum(oeq, KPAD), 16)],
                                  idxv, mask=meq)
            ogt = ogt + plsc.all_reduce_population_count(mgt)[0]
            oeq = oeq + plsc.all_reduce_population_count(meq)[0]
            return ogt, oeq
        n_gt, _ = lax.fori_loop(0, nvec, cb, (jnp.int32(0), jnp.int32(0)))

        def lb(v, _):
            p16 = v * 16 + iota
            valid = p16 < n_gt
            gidx = jnp.where(valid, gt_idx[pl.ds(v * 16, 16)], 0)
            keys = plsc.load_gather(srow, [gidx])
            gkeys[pl.ds(v * 16, 16)] = jnp.where(valid, keys, jnp.float32(-1.0))
            gidxs[pl.ds(v * 16, 16)] = jnp.where(valid, gidx, N + p16)
            return 0
        lax.fori_loop(0, KPAD // 16, lb, 0)

        def rt(t, _):
            kt = gkeys[pl.ds(t * 16, 16)]
            it = gidxs[pl.ds(t * 16, 16)]

            def rs(sv, cnt):
                kvec = gkeys[pl.ds(sv * 16, 16)]
                ivec = gidxs[pl.ds(sv * 16, 16)]
                for l in range(16):
                    kj = kvec[l]
                    ij = ivec[l]
                    beat = (kj > kt) | ((kj == kt) & (ij < it))
                    cnt = cnt + jnp.where(beat, 1, 0)
                return cnt
            rank = lax.fori_loop(0, KPAD // 16, rs,
                                 jnp.zeros((16,), jnp.int32))
            plsc.store_scatter(sorted_idx, [rank], it)
            return 0
        lax.fori_loop(0, KPAD // 16, rt, 0)

        def eb(v, _):
            p16 = v * 16 + iota
            cur = sorted_idx[pl.ds(v * 16, 16)]
            ev = plsc.load_gather(eq_idx, [jnp.clip(p16 - n_gt, 0, KPAD - 1)])
            sorted_idx[pl.ds(v * 16, 16)] = jnp.where(p16 >= n_gt, ev, cur)
            return 0
        lax.fori_loop(0, KPAD // 16, eb, 0)

        def ob(v, _):
            sic = jnp.clip(sorted_idx[pl.ds(v * 16, 16)], 0, N - 1)
            out_s[pl.ds(v * 16, 16)] = plsc.load_gather(srow, [sic])
            out_l[pl.ds(v * 16, 16)] = plsc.load_gather(lrow, [sic])
            return 0
        lax.fori_loop(0, KPAD // 16, ob, 0)

        szlo = sz[pl.ds(0, 16)]
        szhi = sz[pl.ds(16, 16)]
        szsel = jnp.where(b < 8, szlo, szhi)
        lane = jnp.broadcast_to((2 * b) % 16, (16,))
        hf = szsel.at[lane].get(mode="promise_in_bounds")[0]
        wf = szsel.at[lane + 1].get(mode="promise_in_bounds")[0]
        half = jnp.float32(0.5)

        def bb(v, _):
            p16 = v * 16 + iota
            kq = p16 >> 2
            comp = p16 & 3
            base = comp & 1
            qv = jnp.clip(plsc.load_gather(sorted_idx, [kq]), 0, N - 1)
            ctr = plsc.load_gather(brows, [qv * 4 + base])
            ext = plsc.load_gather(brows, [qv * 4 + base + 2])
            sgn = jnp.where(comp >= 2, half, -half)
            scl = jnp.where(base == 1, hf, wf)
            outb[pl.ds(v * 16, 16)] = (ctr + sgn * ext) * scl
            return 0
        lax.fori_loop(0, KPAD * 4 // 16, bb, 0)

        pltpu.sync_copy(outb, boxes_out.at[b])
        pltpu.sync_copy(out_s, scores_out.at[b])
        pltpu.sync_copy(out_l, labels_out.at[b])


def _topk_sc(scores, labels, boxes_flat, sizes):
    mesh = plsc.VectorSubcoreMesh(core_axis_name="c", subcore_axis_name="s")
    f = pl.kernel(
        _topk_body,
        out_type=[
            jax.ShapeDtypeStruct((B, KPAD * 4), jnp.float32),
            jax.ShapeDtypeStruct((B, KPAD), jnp.float32),
            jax.ShapeDtypeStruct((B, KPAD), jnp.int32),
        ],
        mesh=mesh,
        compiler_params=pltpu.CompilerParams(needs_layout_passes=False),
        scratch_types=[
            pltpu.VMEM((N,), jnp.float32),
            pltpu.VMEM((N,), jnp.int32),
            pltpu.VMEM((4096,), jnp.int32),
            pltpu.VMEM((272,), jnp.int32),
            pltpu.VMEM((320,), jnp.int32),
            pltpu.VMEM((KPAD + 32,), jnp.int32),
            pltpu.VMEM((KPAD,), jnp.float32),
            pltpu.VMEM((KPAD,), jnp.int32),
            pltpu.VMEM((KPAD,), jnp.int32),
            pltpu.VMEM((KPAD,), jnp.float32),
            pltpu.VMEM((KPAD,), jnp.int32),
            pltpu.VMEM((N * 4,), jnp.float32),
            pltpu.VMEM((KPAD * 4,), jnp.float32),
            pltpu.VMEM((2 * B,), jnp.float32),
        ],
    )
    return f(scores, labels, boxes_flat, sizes)


def kernel(pred_logits, pred_boxes, orig_target_sizes):
    scores, labels = _scores_labels(pred_logits)
    sizes_f = orig_target_sizes.astype(jnp.float32).reshape(2 * B)
    bflat, s, l = _topk_sc(scores, labels, pred_boxes.reshape(B, N * 4),
                           sizes_f)
    boxes = bflat.reshape(B, KPAD, 4)[:, :NUM_TOP]
    return boxes, s[:, :NUM_TOP], l[:, :NUM_TOP]


# speedup vs baseline: 2.1762x; 1.0716x over previous
import functools

import jax
import jax.numpy as jnp
from jax import lax
from jax.experimental import pallas as pl
from jax.experimental.pallas import tpu as pltpu
from jax.experimental.pallas import tpu_sc as plsc

NUM_TOP = 300
B, N, C = 16, 20000, 80
NBLK = 10000
KPAD = 304
NGAT = 384


def _reduce_body(x_ref, s_ref, l_ref):
    x = x_ref[0]
    s = jax.nn.sigmoid(x)
    st = s.T
    s_ref[0, 0] = jnp.max(st, axis=0)
    l_ref[0, 0] = jnp.argmax(st, axis=0).astype(jnp.int32)


def _scores_labels(pred_logits):
    nblocks = N // NBLK
    grid = (B, nblocks)
    s, l = pl.pallas_call(
        _reduce_body,
        grid=grid,
        in_specs=[pl.BlockSpec((1, NBLK, C), lambda b, i: (b, i, 0))],
        out_specs=[
            pl.BlockSpec((1, 1, NBLK), lambda b, i: (b * nblocks + i, 0, 0)),
            pl.BlockSpec((1, 1, NBLK), lambda b, i: (b * nblocks + i, 0, 0)),
        ],
        out_shape=[
            jax.ShapeDtypeStruct((B * nblocks, 1, NBLK), jnp.float32),
            jax.ShapeDtypeStruct((B * nblocks, 1, NBLK), jnp.int32),
        ],
    )(pred_logits)
    return s.reshape(B, N), l.reshape(B, N)


def _topk_body(scores_hbm, labels_hbm, boxes_hbm, sizes_hbm,
               boxes_out, scores_out, labels_out,
               srow, lrow, hist, totals, gt_idx, eq_idx, gkeys, gidxs,
               sorted_idx, out_s, out_l, brows, outb, sz):
    iota = lax.iota(jnp.int32, 16)
    wid = lax.axis_index("s") * 2 + lax.axis_index("c")
    b = wid
    nvec = N // 16

    @pl.when(wid < B)
    def _():
        pltpu.sync_copy(scores_hbm.at[b], srow)
        pltpu.sync_copy(labels_hbm.at[b], lrow)
        pltpu.sync_copy(boxes_hbm.at[b], brows)
        pltpu.sync_copy(sizes_hbm, sz)

        ones = jnp.ones((16,), jnp.int32)

        def select_pass(p, prefix, k_rem):
            sh = 24 - 8 * p

            def zh(i, _):
                hist[pl.ds(i * 16, 16)] = jnp.zeros((16,), jnp.int32)
                return 0
            lax.fori_loop(0, 256, zh, 0)

            def hb(i, _):
                kv = lax.bitcast_convert_type(srow[pl.ds(i * 16, 16)], jnp.int32)
                d = ((kv >> sh) & 0xFF).astype(jnp.int32)
                if p == 0:
                    match = iota >= 0
                else:
                    match = (kv >> (sh + 8)) == prefix
                hidx = d * 16 + iota
                cur = plsc.load_gather(hist, [hidx])
                plsc.store_scatter(hist, [hidx],
                                   cur + jnp.where(match, 1, 0))
                return 0
            lax.fori_loop(0, nvec, hb, 0)

            def tb(g, _):
                acc = jnp.zeros((16,), jnp.int32)
                for j in range(16):
                    acc = acc + plsc.load_gather(hist, [(g * 16 + iota) * 16 + j])
                totals[pl.ds(g * 16, 16)] = acc
                return 0
            lax.fori_loop(0, 16, tb, 0)

            def sb(gi, carry):
                cum, cnt = carry
                g = 15 - gi
                tg = totals[pl.ds(g * 16, 16)]
                sfx = lax.rev(plsc.cumsum(lax.rev(tg, (0,))), (0,)) + cum
                cnt = cnt + plsc.all_reduce_population_count(sfx >= k_rem)[0]
                cum = cum + jnp.sum(tg)
                return cum, cnt
            _, pop = lax.fori_loop(0, 16, sb, (jnp.int32(0), jnp.int32(0)))
            dstar = pop - 1

            def s2b(g, acc):
                tg = totals[pl.ds(g * 16, 16)]
                return acc + jnp.sum(jnp.where((g * 16 + iota) >= dstar, tg, 0))
            s_dstar = lax.fori_loop(0, 16, s2b, jnp.int32(0))
            t_dstar = totals[pl.ds(dstar, 16)][0]
            k_rem = k_rem - (s_dstar - t_dstar)
            prefix = (prefix << 8) | dstar
            return prefix, k_rem

        prefix = jnp.int32(0)
        k_rem = jnp.int32(NUM_TOP)
        for p in range(4):
            prefix, k_rem = select_pass(p, prefix, k_rem)

        def cb(i, carry):
            ogt, oeq = carry
            kv = lax.bitcast_convert_type(srow[pl.ds(i * 16, 16)], jnp.int32)
            idxv = i * 16 + iota
            mgt = kv > prefix
            meq = kv == prefix
            plsc.store_compressed(gt_idx.at[pl.ds(ogt, 16)], idxv, mask=mgt)
            plsc.store_compressed(eq_idx.at[pl.ds(jnp.minimum(oeq, KPAD), 16)],
                                  idxv, mask=meq)
            ogt = ogt + plsc.all_reduce_population_count(mgt)[0]
            oeq = oeq + plsc.all_reduce_population_count(meq)[0]
            return ogt, oeq
        n_gt, _ = lax.fori_loop(0, nvec, cb, (jnp.int32(0), jnp.int32(0)))

        def lb(v, _):
            p16 = v * 16 + iota
            valid = p16 < n_gt
            gidx = jnp.where(valid, gt_idx[pl.ds(v * 16, 16)], 0)
            keys = plsc.load_gather(srow, [gidx])
            gkeys[pl.ds(v * 16, 16)] = jnp.where(valid, keys, jnp.float32(-1.0))
            gidxs[pl.ds(v * 16, 16)] = jnp.where(valid, gidx, N + p16)
            return 0
        lax.fori_loop(0, KPAD // 16, lb, 0)

        def rt(t, _):
            kt = gkeys[pl.ds(t * 16, 16)]
            it = gidxs[pl.ds(t * 16, 16)]

            def rs(sv, cnt):
                kvec = gkeys[pl.ds(sv * 16, 16)]
                ivec = gidxs[pl.ds(sv * 16, 16)]
                for l in range(16):
                    kj = kvec[l]
                    ij = ivec[l]
                    beat = (kj > kt) | ((kj == kt) & (ij < it))
                    cnt = cnt + jnp.where(beat, 1, 0)
                return cnt
            rank = lax.fori_loop(0, KPAD // 16, rs,
                                 jnp.zeros((16,), jnp.int32))
            plsc.store_scatter(sorted_idx, [rank], it)
            return 0
        lax.fori_loop(0, KPAD // 16, rt, 0)

        def eb(v, _):
            p16 = v * 16 + iota
            cur = sorted_idx[pl.ds(v * 16, 16)]
            ev = plsc.load_gather(eq_idx, [jnp.clip(p16 - n_gt, 0, KPAD - 1)])
            sorted_idx[pl.ds(v * 16, 16)] = jnp.where(p16 >= n_gt, ev, cur)
            return 0
        lax.fori_loop(0, KPAD // 16, eb, 0)

        def ob(v, _):
            sic = jnp.clip(sorted_idx[pl.ds(v * 16, 16)], 0, N - 1)
            out_s[pl.ds(v * 16, 16)] = plsc.load_gather(srow, [sic])
            out_l[pl.ds(v * 16, 16)] = plsc.load_gather(lrow, [sic])
            return 0
        lax.fori_loop(0, KPAD // 16, ob, 0)

        szlo = sz[pl.ds(0, 16)]
        szhi = sz[pl.ds(16, 16)]
        szsel = jnp.where(b < 8, szlo, szhi)
        lane = jnp.broadcast_to((2 * b) % 16, (16,))
        hf = szsel.at[lane].get(mode="promise_in_bounds")[0]
        wf = szsel.at[lane + 1].get(mode="promise_in_bounds")[0]
        half = jnp.float32(0.5)

        def bb(v, _):
            p16 = v * 16 + iota
            kq = p16 >> 2
            comp = p16 & 3
            base = comp & 1
            qv = jnp.clip(plsc.load_gather(sorted_idx, [kq]), 0, N - 1)
            ctr = plsc.load_gather(brows, [qv * 4 + base])
            ext = plsc.load_gather(brows, [qv * 4 + base + 2])
            sgn = jnp.where(comp >= 2, half, -half)
            scl = jnp.where(base == 1, hf, wf)
            outb[pl.ds(v * 16, 16)] = (ctr + sgn * ext) * scl
            return 0
        lax.fori_loop(0, KPAD * 4 // 16, bb, 0)

        pltpu.sync_copy(outb, boxes_out.at[b])
        pltpu.sync_copy(out_s, scores_out.at[b])
        pltpu.sync_copy(out_l, labels_out.at[b])


def _topk_sc(scores, labels, boxes_flat, sizes):
    mesh = plsc.VectorSubcoreMesh(core_axis_name="c", subcore_axis_name="s")
    f = pl.kernel(
        _topk_body,
        out_type=[
            jax.ShapeDtypeStruct((B, KPAD * 4), jnp.float32),
            jax.ShapeDtypeStruct((B, KPAD), jnp.float32),
            jax.ShapeDtypeStruct((B, KPAD), jnp.int32),
        ],
        mesh=mesh,
        compiler_params=pltpu.CompilerParams(needs_layout_passes=False),
        scratch_types=[
            pltpu.VMEM((N,), jnp.float32),
            pltpu.VMEM((N,), jnp.int32),
            pltpu.VMEM((4096,), jnp.int32),
            pltpu.VMEM((272,), jnp.int32),
            pltpu.VMEM((320,), jnp.int32),
            pltpu.VMEM((KPAD + 32,), jnp.int32),
            pltpu.VMEM((KPAD,), jnp.float32),
            pltpu.VMEM((KPAD,), jnp.int32),
            pltpu.VMEM((KPAD,), jnp.int32),
            pltpu.VMEM((KPAD,), jnp.float32),
            pltpu.VMEM((KPAD,), jnp.int32),
            pltpu.VMEM((N * 4,), jnp.float32),
            pltpu.VMEM((KPAD * 4,), jnp.float32),
            pltpu.VMEM((2 * B,), jnp.float32),
        ],
    )
    return f(scores, labels, boxes_flat, sizes)


def kernel(pred_logits, pred_boxes, orig_target_sizes):
    scores, labels = _scores_labels(pred_logits)
    sizes_f = orig_target_sizes.astype(jnp.float32).reshape(2 * B)
    bflat, s, l = _topk_sc(scores, labels, pred_boxes.reshape(B, N * 4),
                           sizes_f)
    boxes = bflat.reshape(B, KPAD, 4)[:, :NUM_TOP]
    return boxes, s[:, :NUM_TOP], l[:, :NUM_TOP]


# speedup vs baseline: 2.2959x; 1.0550x over previous
import functools

import jax
import jax.numpy as jnp
from jax import lax
from jax.experimental import pallas as pl
from jax.experimental.pallas import tpu as pltpu
from jax.experimental.pallas import tpu_sc as plsc

NUM_TOP = 300
B, N, C = 16, 20000, 80
NBLK = 20000
KPAD = 304
NGAT = 384


def _reduce_body(x_ref, s_ref, l_ref):
    x = x_ref[0]
    s = jax.nn.sigmoid(x)
    st = s.T
    s_ref[0, 0] = jnp.max(st, axis=0)
    l_ref[0, 0] = jnp.argmax(st, axis=0).astype(jnp.int32)


def _scores_labels(pred_logits):
    nblocks = N // NBLK
    grid = (B, nblocks)
    s, l = pl.pallas_call(
        _reduce_body,
        grid=grid,
        in_specs=[pl.BlockSpec((1, NBLK, C), lambda b, i: (b, i, 0))],
        out_specs=[
            pl.BlockSpec((1, 1, NBLK), lambda b, i: (b * nblocks + i, 0, 0)),
            pl.BlockSpec((1, 1, NBLK), lambda b, i: (b * nblocks + i, 0, 0)),
        ],
        out_shape=[
            jax.ShapeDtypeStruct((B * nblocks, 1, NBLK), jnp.float32),
            jax.ShapeDtypeStruct((B * nblocks, 1, NBLK), jnp.int32),
        ],
    )(pred_logits)
    return s.reshape(B, N), l.reshape(B, N)


def _topk_body(scores_hbm, labels_hbm, boxes_hbm, sizes_hbm,
               boxes_out, scores_out, labels_out,
               srow, lrow, hist, totals, gt_idx, eq_idx, gkeys, gidxs,
               sorted_idx, out_s, out_l, brows, outb, sz):
    iota = lax.iota(jnp.int32, 16)
    wid = lax.axis_index("s") * 2 + lax.axis_index("c")
    b = wid
    nvec = N // 16

    @pl.when(wid < B)
    def _():
        pltpu.sync_copy(scores_hbm.at[b], srow)
        pltpu.sync_copy(labels_hbm.at[b], lrow)
        pltpu.sync_copy(boxes_hbm.at[b], brows)
        pltpu.sync_copy(sizes_hbm, sz)

        ones = jnp.ones((16,), jnp.int32)

        def select_pass(p, prefix, k_rem):
            sh = 24 - 8 * p

            def zh(i, _):
                hist[pl.ds(i * 16, 16)] = jnp.zeros((16,), jnp.int32)
                return 0
            lax.fori_loop(0, 256, zh, 0)

            def hb(i, _):
                kv = lax.bitcast_convert_type(srow[pl.ds(i * 16, 16)], jnp.int32)
                d = ((kv >> sh) & 0xFF).astype(jnp.int32)
                if p == 0:
                    match = iota >= 0
                else:
                    match = (kv >> (sh + 8)) == prefix
                hidx = d * 16 + iota
                cur = plsc.load_gather(hist, [hidx])
                plsc.store_scatter(hist, [hidx],
                                   cur + jnp.where(match, 1, 0))
                return 0
            lax.fori_loop(0, nvec, hb, 0)

            def tb(g, _):
                acc = jnp.zeros((16,), jnp.int32)
                for j in range(16):
                    acc = acc + plsc.load_gather(hist, [(g * 16 + iota) * 16 + j])
                totals[pl.ds(g * 16, 16)] = acc
                return 0
            lax.fori_loop(0, 16, tb, 0)

            def sb(gi, carry):
                cum, cnt = carry
                g = 15 - gi
                tg = totals[pl.ds(g * 16, 16)]
                sfx = lax.rev(plsc.cumsum(lax.rev(tg, (0,))), (0,)) + cum
                cnt = cnt + plsc.all_reduce_population_count(sfx >= k_rem)[0]
                cum = cum + jnp.sum(tg)
                return cum, cnt
            _, pop = lax.fori_loop(0, 16, sb, (jnp.int32(0), jnp.int32(0)))
            dstar = pop - 1

            def s2b(g, acc):
                tg = totals[pl.ds(g * 16, 16)]
                return acc + jnp.sum(jnp.where((g * 16 + iota) >= dstar, tg, 0))
            s_dstar = lax.fori_loop(0, 16, s2b, jnp.int32(0))
            t_dstar = totals[pl.ds(dstar, 16)][0]
            k_rem = k_rem - (s_dstar - t_dstar)
            prefix = (prefix << 8) | dstar
            return prefix, k_rem

        prefix = jnp.int32(0)
        k_rem = jnp.int32(NUM_TOP)
        for p in range(4):
            prefix, k_rem = select_pass(p, prefix, k_rem)

        def cb(i, carry):
            ogt, oeq = carry
            kv = lax.bitcast_convert_type(srow[pl.ds(i * 16, 16)], jnp.int32)
            idxv = i * 16 + iota
            mgt = kv > prefix
            meq = kv == prefix
            plsc.store_compressed(gt_idx.at[pl.ds(ogt, 16)], idxv, mask=mgt)
            plsc.store_compressed(eq_idx.at[pl.ds(jnp.minimum(oeq, KPAD), 16)],
                                  idxv, mask=meq)
            ogt = ogt + plsc.all_reduce_population_count(mgt)[0]
            oeq = oeq + plsc.all_reduce_population_count(meq)[0]
            return ogt, oeq
        n_gt, _ = lax.fori_loop(0, nvec, cb, (jnp.int32(0), jnp.int32(0)))

        def lb(v, _):
            p16 = v * 16 + iota
            valid = p16 < n_gt
            gidx = jnp.where(valid, gt_idx[pl.ds(v * 16, 16)], 0)
            keys = plsc.load_gather(srow, [gidx])
            gkeys[pl.ds(v * 16, 16)] = jnp.where(valid, keys, jnp.float32(-1.0))
            gidxs[pl.ds(v * 16, 16)] = jnp.where(valid, gidx, N + p16)
            return 0
        lax.fori_loop(0, KPAD // 16, lb, 0)

        def rt(t, _):
            kt = gkeys[pl.ds(t * 16, 16)]
            it = gidxs[pl.ds(t * 16, 16)]

            def rs(sv, cnt):
                kvec = gkeys[pl.ds(sv * 16, 16)]
                ivec = gidxs[pl.ds(sv * 16, 16)]
                for l in range(16):
                    kj = kvec[l]
                    ij = ivec[l]
                    beat = (kj > kt) | ((kj == kt) & (ij < it))
                    cnt = cnt + jnp.where(beat, 1, 0)
                return cnt
            rank = lax.fori_loop(0, KPAD // 16, rs,
                                 jnp.zeros((16,), jnp.int32))
            plsc.store_scatter(sorted_idx, [rank], it)
            return 0
        lax.fori_loop(0, KPAD // 16, rt, 0)

        def eb(v, _):
            p16 = v * 16 + iota
            cur = sorted_idx[pl.ds(v * 16, 16)]
            ev = plsc.load_gather(eq_idx, [jnp.clip(p16 - n_gt, 0, KPAD - 1)])
            sorted_idx[pl.ds(v * 16, 16)] = jnp.where(p16 >= n_gt, ev, cur)
            return 0
        lax.fori_loop(0, KPAD // 16, eb, 0)

        def ob(v, _):
            sic = jnp.clip(sorted_idx[pl.ds(v * 16, 16)], 0, N - 1)
            out_s[pl.ds(v * 16, 16)] = plsc.load_gather(srow, [sic])
            out_l[pl.ds(v * 16, 16)] = plsc.load_gather(lrow, [sic])
            return 0
        lax.fori_loop(0, KPAD // 16, ob, 0)

        szlo = sz[pl.ds(0, 16)]
        szhi = sz[pl.ds(16, 16)]
        szsel = jnp.where(b < 8, szlo, szhi)
        lane = jnp.broadcast_to((2 * b) % 16, (16,))
        hf = szsel.at[lane].get(mode="promise_in_bounds")[0]
        wf = szsel.at[lane + 1].get(mode="promise_in_bounds")[0]
        half = jnp.float32(0.5)

        def bb(v, _):
            p16 = v * 16 + iota
            kq = p16 >> 2
            comp = p16 & 3
            base = comp & 1
            qv = jnp.clip(plsc.load_gather(sorted_idx, [kq]), 0, N - 1)
            ctr = plsc.load_gather(brows, [qv * 4 + base])
            ext = plsc.load_gather(brows, [qv * 4 + base + 2])
            sgn = jnp.where(comp >= 2, half, -half)
            scl = jnp.where(base == 1, hf, wf)
            outb[pl.ds(v * 16, 16)] = (ctr + sgn * ext) * scl
            return 0
        lax.fori_loop(0, KPAD * 4 // 16, bb, 0)

        pltpu.sync_copy(outb, boxes_out.at[b])
        pltpu.sync_copy(out_s, scores_out.at[b])
        pltpu.sync_copy(out_l, labels_out.at[b])


def _topk_sc(scores, labels, boxes_flat, sizes):
    mesh = plsc.VectorSubcoreMesh(core_axis_name="c", subcore_axis_name="s")
    f = pl.kernel(
        _topk_body,
        out_type=[
            jax.ShapeDtypeStruct((B, KPAD * 4), jnp.float32),
            jax.ShapeDtypeStruct((B, KPAD), jnp.float32),
            jax.ShapeDtypeStruct((B, KPAD), jnp.int32),
        ],
        mesh=mesh,
        compiler_params=pltpu.CompilerParams(needs_layout_passes=False),
        scratch_types=[
            pltpu.VMEM((N,), jnp.float32),
            pltpu.VMEM((N,), jnp.int32),
            pltpu.VMEM((4096,), jnp.int32),
            pltpu.VMEM((272,), jnp.int32),
            pltpu.VMEM((320,), jnp.int32),
            pltpu.VMEM((KPAD + 32,), jnp.int32),
            pltpu.VMEM((KPAD,), jnp.float32),
            pltpu.VMEM((KPAD,), jnp.int32),
            pltpu.VMEM((KPAD,), jnp.int32),
            pltpu.VMEM((KPAD,), jnp.float32),
            pltpu.VMEM((KPAD,), jnp.int32),
            pltpu.VMEM((N * 4,), jnp.float32),
            pltpu.VMEM((KPAD * 4,), jnp.float32),
            pltpu.VMEM((2 * B,), jnp.float32),
        ],
    )
    return f(scores, labels, boxes_flat, sizes)


def kernel(pred_logits, pred_boxes, orig_target_sizes):
    scores, labels = _scores_labels(pred_logits)
    sizes_f = orig_target_sizes.astype(jnp.float32).reshape(2 * B)
    bflat, s, l = _topk_sc(scores, labels, pred_boxes.reshape(B, N * 4),
                           sizes_f)
    boxes = bflat.reshape(B, KPAD, 4)[:, :NUM_TOP]
    return boxes, s[:, :NUM_TOP], l[:, :NUM_TOP]


# speedup vs baseline: 2.3526x; 1.0247x over previous
import functools

import jax
import jax.numpy as jnp
from jax import lax
from jax.experimental import pallas as pl
from jax.experimental.pallas import tpu as pltpu
from jax.experimental.pallas import tpu_sc as plsc

NUM_TOP = 300
B, N, C = 16, 20000, 80
NBLK = 20000
BB = 2
KPAD = 304
NGAT = 384


def _reduce_body(x_ref, s_ref, l_ref):
    for j in range(BB):
        st = jax.nn.sigmoid(x_ref[j]).T
        s_ref[j, 0] = jnp.max(st, axis=0)
        l_ref[j, 0] = jnp.argmax(st, axis=0).astype(jnp.int32)


def _scores_labels(pred_logits):
    grid = (B // BB,)
    s, l = pl.pallas_call(
        _reduce_body,
        grid=grid,
        in_specs=[pl.BlockSpec((BB, NBLK, C), lambda i: (i, 0, 0))],
        out_specs=[
            pl.BlockSpec((BB, 1, NBLK), lambda i: (i, 0, 0)),
            pl.BlockSpec((BB, 1, NBLK), lambda i: (i, 0, 0)),
        ],
        out_shape=[
            jax.ShapeDtypeStruct((B, 1, NBLK), jnp.float32),
            jax.ShapeDtypeStruct((B, 1, NBLK), jnp.int32),
        ],
    )(pred_logits)
    return s.reshape(B, N), l.reshape(B, N)


def _topk_body(scores_hbm, labels_hbm, boxes_hbm, sizes_hbm,
               boxes_out, scores_out, labels_out,
               srow, lrow, hist, totals, gt_idx, eq_idx, gkeys, gidxs,
               sorted_idx, out_s, out_l, brows, outb, sz):
    iota = lax.iota(jnp.int32, 16)
    wid = lax.axis_index("s") * 2 + lax.axis_index("c")
    b = wid
    nvec = N // 16

    @pl.when(wid < B)
    def _():
        pltpu.sync_copy(scores_hbm.at[b], srow)
        pltpu.sync_copy(labels_hbm.at[b], lrow)
        pltpu.sync_copy(sizes_hbm, sz)

        ones = jnp.ones((16,), jnp.int32)

        def select_pass(p, prefix, k_rem):
            sh = 24 - 8 * p

            def zh(i, _):
                hist[pl.ds(i * 16, 16)] = jnp.zeros((16,), jnp.int32)
                return 0
            lax.fori_loop(0, 256, zh, 0)

            def hb(i, _):
                for u in range(8):
                    kv = lax.bitcast_convert_type(
                        srow[pl.ds((i * 8 + u) * 16, 16)], jnp.int32)
                    d = ((kv >> sh) & 0xFF).astype(jnp.int32)
                    if p == 0:
                        match = iota >= 0
                    else:
                        match = (kv >> (sh + 8)) == prefix
                    hidx = d * 16 + iota
                    cur = plsc.load_gather(hist, [hidx])
                    plsc.store_scatter(hist, [hidx],
                                       cur + jnp.where(match, 1, 0))
                return 0
            lax.fori_loop(0, nvec // 8, hb, 0)

            def tb(g, _):
                acc = jnp.zeros((16,), jnp.int32)
                for j in range(16):
                    acc = acc + plsc.load_gather(hist, [(g * 16 + iota) * 16 + j])
                totals[pl.ds(g * 16, 16)] = acc
                return 0
            lax.fori_loop(0, 16, tb, 0)

            def sb(gi, carry):
                cum, cnt = carry
                g = 15 - gi
                tg = totals[pl.ds(g * 16, 16)]
                sfx = lax.rev(plsc.cumsum(lax.rev(tg, (0,))), (0,)) + cum
                cnt = cnt + plsc.all_reduce_population_count(sfx >= k_rem)[0]
                cum = cum + jnp.sum(tg)
                return cum, cnt
            _, pop = lax.fori_loop(0, 16, sb, (jnp.int32(0), jnp.int32(0)))
            dstar = pop - 1

            def s2b(g, acc):
                tg = totals[pl.ds(g * 16, 16)]
                return acc + jnp.sum(jnp.where((g * 16 + iota) >= dstar, tg, 0))
            s_dstar = lax.fori_loop(0, 16, s2b, jnp.int32(0))
            t_dstar = totals[pl.ds(dstar, 16)][0]
            k_rem = k_rem - (s_dstar - t_dstar)
            prefix = (prefix << 8) | dstar
            return prefix, k_rem

        prefix = jnp.int32(0)
        k_rem = jnp.int32(NUM_TOP)
        for p in range(4):
            prefix, k_rem = select_pass(p, prefix, k_rem)

        def cb(i, carry):
            ogt, oeq = carry
            for u in range(8):
                kv = lax.bitcast_convert_type(
                    srow[pl.ds((i * 8 + u) * 16, 16)], jnp.int32)
                idxv = (i * 8 + u) * 16 + iota
                mgt = kv > prefix
                meq = kv == prefix
                plsc.store_compressed(gt_idx.at[pl.ds(ogt, 16)], idxv,
                                      mask=mgt)
                plsc.store_compressed(
                    eq_idx.at[pl.ds(jnp.minimum(oeq, KPAD), 16)],
                    idxv, mask=meq)
                ogt = ogt + plsc.all_reduce_population_count(mgt)[0]
                oeq = oeq + plsc.all_reduce_population_count(meq)[0]
            return ogt, oeq
        n_gt, _ = lax.fori_loop(0, nvec // 8, cb,
                                (jnp.int32(0), jnp.int32(0)))

        def lb(v, _):
            p16 = v * 16 + iota
            valid = p16 < n_gt
            gidx = jnp.where(valid, gt_idx[pl.ds(v * 16, 16)], 0)
            keys = plsc.load_gather(srow, [gidx])
            gkeys[pl.ds(v * 16, 16)] = jnp.where(valid, keys, jnp.float32(-1.0))
            gidxs[pl.ds(v * 16, 16)] = jnp.where(valid, gidx, N + p16)
            return 0
        lax.fori_loop(0, KPAD // 16, lb, 0)

        def rt(t, _):
            kt = gkeys[pl.ds(t * 16, 16)]
            it = gidxs[pl.ds(t * 16, 16)]

            def rs(sv, cnt):
                kvec = gkeys[pl.ds(sv * 16, 16)]
                ivec = gidxs[pl.ds(sv * 16, 16)]
                for l in range(16):
                    kj = kvec[l]
                    ij = ivec[l]
                    beat = (kj > kt) | ((kj == kt) & (ij < it))
                    cnt = cnt + jnp.where(beat, 1, 0)
                return cnt
            rank = lax.fori_loop(0, KPAD // 16, rs,
                                 jnp.zeros((16,), jnp.int32))
            plsc.store_scatter(sorted_idx, [rank], it)
            return 0
        lax.fori_loop(0, KPAD // 16, rt, 0)

        def eb(v, _):
            p16 = v * 16 + iota
            cur = sorted_idx[pl.ds(v * 16, 16)]
            ev = plsc.load_gather(eq_idx, [jnp.clip(p16 - n_gt, 0, KPAD - 1)])
            sorted_idx[pl.ds(v * 16, 16)] = jnp.where(p16 >= n_gt, ev, cur)
            return 0
        lax.fori_loop(0, KPAD // 16, eb, 0)

        def ob(v, _):
            sic = jnp.clip(sorted_idx[pl.ds(v * 16, 16)], 0, N - 1)
            out_s[pl.ds(v * 16, 16)] = plsc.load_gather(srow, [sic])
            out_l[pl.ds(v * 16, 16)] = plsc.load_gather(lrow, [sic])
            return 0
        lax.fori_loop(0, KPAD // 16, ob, 0)

        szlo = sz[pl.ds(0, 16)]
        szhi = sz[pl.ds(16, 16)]
        szsel = jnp.where(b < 8, szlo, szhi)
        lane = jnp.broadcast_to((2 * b) % 16, (16,))
        hf = szsel.at[lane].get(mode="promise_in_bounds")[0]
        wf = szsel.at[lane + 1].get(mode="promise_in_bounds")[0]
        half = jnp.float32(0.5)

        def bb(v, _):
            p16 = v * 16 + iota
            kq = p16 >> 2
            comp = p16 & 3
            base = comp & 1
            qv = jnp.clip(plsc.load_gather(sorted_idx, [kq]), 0, N - 1)
            ctr = plsc.load_gather(brows, [qv * 4 + base])
            ext = plsc.load_gather(brows, [qv * 4 + base + 2])
            sgn = jnp.where(comp >= 2, half, -half)
            scl = jnp.where(base == 1, hf, wf)
            outb[pl.ds(v * 16, 16)] = (ctr + sgn * ext) * scl
            return 0
        lax.fori_loop(0, KPAD * 4 // 16, bb, 0)

        pltpu.sync_copy(outb, boxes_out.at[b])
        pltpu.sync_copy(out_s, scores_out.at[b])
        pltpu.sync_copy(out_l, labels_out.at[b])


def _topk_sc(scores, labels, boxes_flat, sizes):
    mesh = plsc.VectorSubcoreMesh(core_axis_name="c", subcore_axis_name="s")
    f = pl.kernel(
        _topk_body,
        out_type=[
            jax.ShapeDtypeStruct((B, KPAD * 4), jnp.float32),
            jax.ShapeDtypeStruct((B, KPAD), jnp.float32),
            jax.ShapeDtypeStruct((B, KPAD), jnp.int32),
        ],
        mesh=mesh,
        compiler_params=pltpu.CompilerParams(needs_layout_passes=False),
        scratch_types=[
            pltpu.VMEM((N,), jnp.float32),
            pltpu.VMEM((N,), jnp.int32),
            pltpu.VMEM((4096,), jnp.int32),
            pltpu.VMEM((272,), jnp.int32),
            pltpu.VMEM((320,), jnp.int32),
            pltpu.VMEM((KPAD + 32,), jnp.int32),
            pltpu.VMEM((KPAD,), jnp.float32),
            pltpu.VMEM((KPAD,), jnp.int32),
            pltpu.VMEM((KPAD,), jnp.int32),
            pltpu.VMEM((KPAD,), jnp.float32),
            pltpu.VMEM((KPAD,), jnp.int32),
            pltpu.VMEM((N * 4,), jnp.float32),
            pltpu.VMEM((KPAD * 4,), jnp.float32),
            pltpu.VMEM((2 * B,), jnp.float32),
        ],
    )
    return f(scores, labels, boxes_flat, sizes)


def kernel(pred_logits, pred_boxes, orig_target_sizes):
    scores, labels = _scores_labels(pred_logits)
    sizes_f = orig_target_sizes.astype(jnp.float32).reshape(2 * B)
    bflat, s, l = _topk_sc(scores, labels, pred_boxes.reshape(B, N * 4),
                           sizes_f)
    boxes = bflat.reshape(B, KPAD, 4)[:, :NUM_TOP]
    return boxes, s[:, :NUM_TOP], l[:, :NUM_TOP]
